# Initial kernel scaffold; baseline (speedup 1.0000x reference)
import jax
import jax.numpy as jnp
from jax.experimental import pallas as pl


def kernel(feat, img, dist_diff, msg_node, alpha, beta, gamma, w1, w2):
    raise NotImplementedError("write your pallas kernel here")


# speedup vs baseline: 7.0291x; 7.0291x over previous
import functools

import jax
import jax.numpy as jnp
from jax import lax
from jax.experimental import pallas as pl
from jax.experimental.pallas import tpu as pltpu
from jax.experimental.pallas import tpu_sc as plsc

D = 32
DI = 16
CHUNK = 128
NW = 32
BIG = 1.0e30


def _take16(v, j):
    idx = jnp.full((16, 1), j, dtype=jnp.int32)
    dnums = lax.GatherDimensionNumbers(
        offset_dims=(), collapsed_slice_dims=(0,), start_index_map=(0,))
    return lax.gather(v, idx, dnums, (1,),
                      mode=lax.GatherScatterMode.PROMISE_IN_BOUNDS)


def _init_body(k, featp_ref, q0_ref, un_ref):
    x = featp_ref[...]
    lane = lax.broadcasted_iota(jnp.int32, x.shape, 1)
    xm = jnp.where(lane < k, x, -BIG)
    m = jnp.max(xm, axis=1, keepdims=True)
    e = jnp.exp(xm - m)
    s = jnp.sum(e, axis=1, keepdims=True)
    q0_ref[...] = e / s
    un_ref[...] = jnp.where(lane < k, m + jnp.log(s) - xm, BIG)


def _wvec(k, shape):
    lane = lax.broadcasted_iota(jnp.int32, shape, 1)
    return jnp.where(lane == k - 1, 10.0,
                     jnp.where(lane < k, 1.0, 0.0)).astype(jnp.float32)


def _update_body(k, log_out, agg_ref, un_ref, out_ref):
    agg = agg_ref[0] + agg_ref[1]
    w = _wvec(k, agg.shape)
    s = jnp.sum(agg * w, axis=1, keepdims=True)
    logit = w * w * agg - w * s - un_ref[...]
    m = jnp.max(logit, axis=1, keepdims=True)
    e = jnp.exp(logit - m)
    z = jnp.sum(e, axis=1, keepdims=True)
    if log_out:
        out_ref[...] = logit - m - jnp.log(z)
    else:
        out_ref[...] = e / z


def _edge_coef_kernel(n, e_total):
    ew = e_total // NW
    nchunks = ew // CHUNK
    mesh = plsc.VectorSubcoreMesh(core_axis_name="c", subcore_axis_name="s", num_cores=2, num_subcores=16)

    @functools.partial(
        pl.kernel,
        out_type=jax.ShapeDtypeStruct((e_total,), jnp.float32),
        mesh=mesh,
        compiler_params=pltpu.CompilerParams(use_tc_tiling_on_sc=False),
        scratch_types=[
            pltpu.VMEM((CHUNK,), jnp.int32),
            pltpu.VMEM((CHUNK,), jnp.int32),
            pltpu.VMEM((CHUNK,), jnp.float32),
            pltpu.VMEM((CHUNK,), jnp.float32),
            [pltpu.VMEM((CHUNK,), jnp.float32)] * 3,
            [pltpu.VMEM((CHUNK,), jnp.float32)] * 3,
            pltpu.VMEM((CHUNK,), jnp.float32),
            pltpu.VMEM((16,), jnp.float32),
            pltpu.SemaphoreType.DMA,
        ],
    )
    def coef_kernel(ch0, ch1, ch2, ein, eout, d0, d1, params, ke_out,
                    einv, eoutv, d0v, d1v, rin, rout, kev, prmv, sem):
        wid = lax.axis_index("c") * 16 + lax.axis_index("s")
        base0 = wid * ew
        pltpu.sync_copy(params, prmv)
        praw = prmv[...]
        pos = lax.iota(jnp.int32, 16)
        cv = jnp.where(pos < 7, 1.0 / (2.0 * praw * praw), jnp.abs(praw))
        a0 = _take16(cv, 0)
        a1 = _take16(cv, 1)
        b = [_take16(cv, 2), _take16(cv, 3), _take16(cv, 4)]
        g0 = _take16(cv, 5)
        g1 = _take16(cv, 6)
        aw1 = _take16(cv, 7)
        aw2 = _take16(cv, 8)
        chans = (ch0, ch1, ch2)

        def body(i, _):
            base = base0 + i * CHUNK
            pltpu.sync_copy(ein.at[pl.ds(base, CHUNK)], einv)
            pltpu.sync_copy(eout.at[pl.ds(base, CHUNK)], eoutv)
            pltpu.sync_copy(d0.at[pl.ds(base, CHUNK)], d0v)
            pltpu.sync_copy(d1.at[pl.ds(base, CHUNK)], d1v)
            cps = []
            for ch in range(3):
                cps.append(pltpu.async_copy(chans[ch].at[einv], rin[ch], sem))
                cps.append(pltpu.async_copy(chans[ch].at[eoutv], rout[ch], sem))
            for cp in cps:
                cp.wait()
            for gidx in range(CHUNK // 16):
                sl = pl.ds(gidx * 16, 16)
                acc = jnp.zeros((16,), jnp.float32)
                for ch in range(3):
                    dd = rin[ch][sl] - rout[ch][sl]
                    acc = acc + dd * dd * b[ch]
                dv0 = d0v[sl]
                dv1 = d1v[sl]
                k1 = aw1 * jnp.exp(-(dv0 * a0 + dv1 * a1) - acc)
                k2 = aw2 * jnp.exp(-(dv0 * g0 + dv1 * g1))
                kev[sl] = k1 + k2
            pltpu.sync_copy(kev, ke_out.at[pl.ds(base, CHUNK)])
            return _

        lax.fori_loop(0, nchunks, body, None)

    return coef_kernel


def _agg_kernel(n, e_total):
    ew = e_total // NW
    nchunks = ew // CHUNK
    rows_per_tile = n // 16
    zc = rows_per_tile // 7
    mesh = plsc.VectorSubcoreMesh(core_axis_name="c", subcore_axis_name="s", num_cores=2, num_subcores=16)

    @functools.partial(
        pl.kernel,
        out_type=jax.ShapeDtypeStruct((2, n, D), jnp.float32),
        mesh=mesh,
        compiler_params=pltpu.CompilerParams(use_tc_tiling_on_sc=False),
        scratch_types=[
            pltpu.VMEM_SHARED((n, D), jnp.float32),
            pltpu.VMEM((CHUNK,), jnp.int32),
            pltpu.VMEM((CHUNK,), jnp.int32),
            pltpu.VMEM((CHUNK,), jnp.float32),
            pltpu.VMEM((CHUNK, D), jnp.float32),
            pltpu.VMEM((zc, D), jnp.float32),
            pltpu.SemaphoreType.DMA,
        ],
    )
    def agg_kernel(qp, ein, eout, ke, agg_out,
                   aggs, einv, eoutv, kev, rows, zbuf, sem):
        cid = lax.axis_index("c")
        sid = lax.axis_index("s")
        wid = cid * 16 + sid
        base0 = wid * ew
        r0 = sid * rows_per_tile

        def zrow(i, _):
            zbuf[i, pl.ds(0, 16)] = jnp.zeros((16,), jnp.float32)
            zbuf[i, pl.ds(16, 16)] = jnp.zeros((16,), jnp.float32)
            return _
        lax.fori_loop(0, zc, zrow, None)

        def zcopy(i, _):
            pltpu.sync_copy(zbuf, aggs.at[pl.ds(r0 + i * zc, zc)])
            return _
        lax.fori_loop(0, 7, zcopy, None)
        plsc.subcore_barrier()

        def body(i, _):
            base = base0 + i * CHUNK
            pltpu.sync_copy(ein.at[pl.ds(base, CHUNK)], einv)
            pltpu.sync_copy(eout.at[pl.ds(base, CHUNK)], eoutv)
            pltpu.sync_copy(ke.at[pl.ds(base, CHUNK)], kev)
            pltpu.async_copy(qp.at[einv], rows, sem).wait()
            for gidx in range(CHUNK // 16):
                kv = kev[pl.ds(gidx * 16, 16)]
                for j in range(16):
                    e = gidx * 16 + j
                    s = _take16(kv, j)
                    rows[e, pl.ds(0, 16)] = rows[e, pl.ds(0, 16)] * s
                    rows[e, pl.ds(16, 16)] = rows[e, pl.ds(16, 16)] * s
            pltpu.sync_copy(rows, aggs.at[eoutv], add=True)
            return _

        lax.fori_loop(0, nchunks, body, None)
        plsc.subcore_barrier()

        def ocopy(i, _):
            pltpu.sync_copy(aggs.at[pl.ds(r0 + i * zc, zc)],
                            agg_out.at[cid, pl.ds(r0 + i * zc, zc)])
            return _
        lax.fori_loop(0, 7, ocopy, None)

    return agg_kernel


def kernel(feat, img, dist_diff, msg_node, alpha, beta, gamma, w1, w2):
    B, K, H, W = feat.shape
    N = B * H * W
    E = msg_node.shape[0]
    MAX_IT = 10

    featp = jnp.pad(feat.reshape(K, N).T, ((0, 0), (0, D - K)))
    imgc = img.reshape(3, N)
    ein = msg_node[:, 0]
    eout = msg_node[:, 1]
    d0 = dist_diff[:, 0]
    d1 = dist_diff[:, 1]
    params = jnp.concatenate([alpha.ravel(), beta.ravel(), gamma.ravel(),
                              w1.ravel(), w2.ravel(),
                              jnp.ones((7,), jnp.float32)])

    RB = 256
    grid = (N // RB,)

    q0, unary = pl.pallas_call(
        functools.partial(_init_body, K),
        grid=grid,
        in_specs=[pl.BlockSpec((RB, D), lambda i: (i, 0))],
        out_specs=[pl.BlockSpec((RB, D), lambda i: (i, 0))] * 2,
        out_shape=[jax.ShapeDtypeStruct((N, D), jnp.float32)] * 2,
    )(featp)

    ke = _edge_coef_kernel(N, E)(imgc[0], imgc[1], imgc[2],
                                 ein, eout, d0, d1, params)

    agg_k = _agg_kernel(N, E)

    def update(log_out, agg2, un):
        return pl.pallas_call(
            functools.partial(_update_body, K, log_out),
            grid=grid,
            in_specs=[pl.BlockSpec((2, RB, D), lambda i: (0, i, 0)),
                      pl.BlockSpec((RB, D), lambda i: (i, 0))],
            out_specs=pl.BlockSpec((RB, D), lambda i: (i, 0)),
            out_shape=jax.ShapeDtypeStruct((N, D), jnp.float32),
        )(agg2, un)

    q = q0
    for it in range(MAX_IT):
        agg2 = agg_k(q, ein, eout, ke)
        q = update(it == MAX_IT - 1, agg2, unary)

    logq = q[:, :K].reshape(B, H, W, K)
    return jnp.transpose(logq, (0, 3, 1, 2))


# speedup vs baseline: 11.9314x; 1.6974x over previous
import functools

import jax
import jax.numpy as jnp
from jax import lax
from jax.experimental import pallas as pl
from jax.experimental.pallas import tpu as pltpu
from jax.experimental.pallas import tpu_sc as plsc

D = 32
DI = 16
CHUNK = 128
NW = 32
BIG = 1.0e30


def _take16(v, j):
    idx = jnp.full((16, 1), j, dtype=jnp.int32)
    dnums = lax.GatherDimensionNumbers(
        offset_dims=(), collapsed_slice_dims=(0,), start_index_map=(0,))
    return lax.gather(v, idx, dnums, (1,),
                      mode=lax.GatherScatterMode.PROMISE_IN_BOUNDS)


def _init_body(k, featp_ref, q0_ref, un_ref):
    x = featp_ref[...]
    lane = lax.broadcasted_iota(jnp.int32, x.shape, 1)
    xm = jnp.where(lane < k, x, -BIG)
    m = jnp.max(xm, axis=1, keepdims=True)
    e = jnp.exp(xm - m)
    s = jnp.sum(e, axis=1, keepdims=True)
    q0_ref[...] = e / s
    un_ref[...] = jnp.where(lane < k, m + jnp.log(s) - xm, BIG)


def _wvec(k, shape):
    lane = lax.broadcasted_iota(jnp.int32, shape, 1)
    return jnp.where(lane == k - 1, 10.0,
                     jnp.where(lane < k, 1.0, 0.0)).astype(jnp.float32)


def _update_body(k, log_out, agg_ref, un_ref, out_ref):
    agg = agg_ref[0] + agg_ref[1]
    w = _wvec(k, agg.shape)
    s = jnp.sum(agg * w, axis=1, keepdims=True)
    logit = w * w * agg - w * s - un_ref[...]
    m = jnp.max(logit, axis=1, keepdims=True)
    e = jnp.exp(logit - m)
    z = jnp.sum(e, axis=1, keepdims=True)
    if log_out:
        out_ref[...] = logit - m - jnp.log(z)
    else:
        out_ref[...] = e / z


def _edge_coef_kernel(n, e_total):
    ew = e_total // NW
    nchunks = ew // CHUNK
    mesh = plsc.VectorSubcoreMesh(core_axis_name="c", subcore_axis_name="s", num_cores=2, num_subcores=16)

    @functools.partial(
        pl.kernel,
        out_type=jax.ShapeDtypeStruct((e_total,), jnp.float32),
        mesh=mesh,
        compiler_params=pltpu.CompilerParams(use_tc_tiling_on_sc=False),
        scratch_types=[
            pltpu.VMEM((CHUNK,), jnp.int32),
            pltpu.VMEM((CHUNK,), jnp.int32),
            pltpu.VMEM((CHUNK,), jnp.float32),
            pltpu.VMEM((CHUNK,), jnp.float32),
            [pltpu.VMEM((CHUNK,), jnp.float32)] * 3,
            [pltpu.VMEM((CHUNK,), jnp.float32)] * 3,
            pltpu.VMEM((CHUNK,), jnp.float32),
            pltpu.VMEM((16,), jnp.float32),
            pltpu.SemaphoreType.DMA,
        ],
    )
    def coef_kernel(ch0, ch1, ch2, ein, eout, d0, d1, params, ke_out,
                    einv, eoutv, d0v, d1v, rin, rout, kev, prmv, sem):
        wid = lax.axis_index("c") * 16 + lax.axis_index("s")
        base0 = wid * ew
        pltpu.sync_copy(params, prmv)
        praw = prmv[...]
        pos = lax.iota(jnp.int32, 16)
        cv = jnp.where(pos < 7, 1.0 / (2.0 * praw * praw), jnp.abs(praw))
        a0 = _take16(cv, 0)
        a1 = _take16(cv, 1)
        b = [_take16(cv, 2), _take16(cv, 3), _take16(cv, 4)]
        g0 = _take16(cv, 5)
        g1 = _take16(cv, 6)
        aw1 = _take16(cv, 7)
        aw2 = _take16(cv, 8)
        chans = (ch0, ch1, ch2)

        def body(i, _):
            base = base0 + i * CHUNK
            pltpu.sync_copy(ein.at[pl.ds(base, CHUNK)], einv)
            pltpu.sync_copy(eout.at[pl.ds(base, CHUNK)], eoutv)
            pltpu.sync_copy(d0.at[pl.ds(base, CHUNK)], d0v)
            pltpu.sync_copy(d1.at[pl.ds(base, CHUNK)], d1v)
            cps = []
            for ch in range(3):
                cps.append(pltpu.async_copy(chans[ch].at[einv], rin[ch], sem))
                cps.append(pltpu.async_copy(chans[ch].at[eoutv], rout[ch], sem))
            for cp in cps:
                cp.wait()
            for gidx in range(CHUNK // 16):
                sl = pl.ds(gidx * 16, 16)
                acc = jnp.zeros((16,), jnp.float32)
                for ch in range(3):
                    dd = rin[ch][sl] - rout[ch][sl]
                    acc = acc + dd * dd * b[ch]
                dv0 = d0v[sl]
                dv1 = d1v[sl]
                k1 = aw1 * jnp.exp(-(dv0 * a0 + dv1 * a1) - acc)
                k2 = aw2 * jnp.exp(-(dv0 * g0 + dv1 * g1))
                kev[sl] = k1 + k2
            pltpu.sync_copy(kev, ke_out.at[pl.ds(base, CHUNK)])
            return _

        lax.fori_loop(0, nchunks, body, None)

    return coef_kernel


def _agg_kernel(n, e_total):
    ew = e_total // NW
    nchunks = ew // CHUNK
    rows_per_tile = n // 16
    zc = rows_per_tile // 7
    mesh = plsc.VectorSubcoreMesh(core_axis_name="c", subcore_axis_name="s", num_cores=2, num_subcores=16)

    @functools.partial(
        pl.kernel,
        out_type=jax.ShapeDtypeStruct((2, n, D), jnp.float32),
        mesh=mesh,
        compiler_params=pltpu.CompilerParams(use_tc_tiling_on_sc=False),
        scratch_types=[
            pltpu.VMEM_SHARED((n, D), jnp.float32),
            [pltpu.VMEM((CHUNK,), jnp.int32)] * 2,
            [pltpu.VMEM((CHUNK,), jnp.int32)] * 2,
            [pltpu.VMEM((CHUNK,), jnp.float32)] * 2,
            [pltpu.VMEM((CHUNK, D), jnp.float32)] * 2,
            pltpu.VMEM((zc, D), jnp.float32),
            [pltpu.SemaphoreType.DMA] * 2,
            [pltpu.SemaphoreType.DMA] * 2,
        ],
    )
    def agg_kernel(qp, ein, eout, ke, agg_out,
                   aggs, einv, eoutv, kev, rows, zbuf, semi, semg):
        cid = lax.axis_index("c")
        sid = lax.axis_index("s")
        wid = cid * 16 + sid
        base0 = wid * ew
        r0 = sid * rows_per_tile

        def zrow(i, _):
            zbuf[i, pl.ds(0, 16)] = jnp.zeros((16,), jnp.float32)
            zbuf[i, pl.ds(16, 16)] = jnp.zeros((16,), jnp.float32)
            return _
        lax.fori_loop(0, zc, zrow, None)

        def zcopy(i, _):
            pltpu.sync_copy(zbuf, aggs.at[pl.ds(r0 + i * zc, zc)])
            return _
        lax.fori_loop(0, 7, zcopy, None)
        plsc.subcore_barrier()

        def fire_idx(i, p):
            base = base0 + i * CHUNK
            pltpu.async_copy(ein.at[pl.ds(base, CHUNK)], einv[p], semi[p])
            pltpu.async_copy(eout.at[pl.ds(base, CHUNK)], eoutv[p], semi[p])
            pltpu.async_copy(ke.at[pl.ds(base, CHUNK)], kev[p], semi[p])

        def wait_idx(p):
            pltpu.make_async_copy(ein.at[pl.ds(0, CHUNK)], einv[p], semi[p]).wait()
            pltpu.make_async_copy(eout.at[pl.ds(0, CHUNK)], eoutv[p], semi[p]).wait()
            pltpu.make_async_copy(ke.at[pl.ds(0, CHUNK)], kev[p], semi[p]).wait()

        def fire_gather(p):
            pltpu.async_copy(qp.at[einv[p]], rows[p], semg[p])

        def wait_gather(p):
            pltpu.make_async_copy(qp.at[einv[p]], rows[p], semg[p]).wait()

        fire_idx(0, 0)
        fire_idx(1, 1)
        wait_idx(0)
        fire_gather(0)

        def step(i, par):
            @pl.when(i + 1 < nchunks)
            def _():
                wait_idx(1 - par)
                fire_gather(1 - par)
            wait_gather(par)
            rw = rows[par]
            kw = kev[par]
            for gidx in range(CHUNK // 16):
                kv = kw[pl.ds(gidx * 16, 16)]
                for j in range(16):
                    e = gidx * 16 + j
                    s = _take16(kv, j)
                    rw[e, pl.ds(0, 16)] = rw[e, pl.ds(0, 16)] * s
                    rw[e, pl.ds(16, 16)] = rw[e, pl.ds(16, 16)] * s
            pltpu.sync_copy(rw, aggs.at[eoutv[par]], add=True)

            @pl.when(i + 2 < nchunks)
            def _():
                fire_idx(i + 2, par)

        def body2(i2, _):
            step(i2 * 2, 0)
            step(i2 * 2 + 1, 1)
            return _

        lax.fori_loop(0, nchunks // 2, body2, None)
        plsc.subcore_barrier()

        def ocopy(i, _):
            pltpu.sync_copy(aggs.at[pl.ds(r0 + i * zc, zc)],
                            agg_out.at[cid, pl.ds(r0 + i * zc, zc)])
            return _
        lax.fori_loop(0, 7, ocopy, None)

    return agg_kernel


def kernel(feat, img, dist_diff, msg_node, alpha, beta, gamma, w1, w2):
    B, K, H, W = feat.shape
    N = B * H * W
    E = msg_node.shape[0]
    MAX_IT = 10

    featp = jnp.pad(feat.reshape(K, N).T, ((0, 0), (0, D - K)))
    imgc = img.reshape(3, N)
    ein = msg_node[:, 0]
    eout = msg_node[:, 1]
    d0 = dist_diff[:, 0]
    d1 = dist_diff[:, 1]
    params = jnp.concatenate([alpha.ravel(), beta.ravel(), gamma.ravel(),
                              w1.ravel(), w2.ravel(),
                              jnp.ones((7,), jnp.float32)])

    RB = 256
    grid = (N // RB,)

    q0, unary = pl.pallas_call(
        functools.partial(_init_body, K),
        grid=grid,
        in_specs=[pl.BlockSpec((RB, D), lambda i: (i, 0))],
        out_specs=[pl.BlockSpec((RB, D), lambda i: (i, 0))] * 2,
        out_shape=[jax.ShapeDtypeStruct((N, D), jnp.float32)] * 2,
    )(featp)

    ke = _edge_coef_kernel(N, E)(imgc[0], imgc[1], imgc[2],
                                 ein, eout, d0, d1, params)

    agg_k = _agg_kernel(N, E)

    def update(log_out, agg2, un):
        return pl.pallas_call(
            functools.partial(_update_body, K, log_out),
            grid=grid,
            in_specs=[pl.BlockSpec((2, RB, D), lambda i: (0, i, 0)),
                      pl.BlockSpec((RB, D), lambda i: (i, 0))],
            out_specs=pl.BlockSpec((RB, D), lambda i: (i, 0)),
            out_shape=jax.ShapeDtypeStruct((N, D), jnp.float32),
        )(agg2, un)

    q = q0
    for it in range(MAX_IT):
        agg2 = agg_k(q, ein, eout, ke)
        q = update(it == MAX_IT - 1, agg2, unary)

    logq = q[:, :K].reshape(B, H, W, K)
    return jnp.transpose(logq, (0, 3, 1, 2))


# speedup vs baseline: 14.8209x; 1.2422x over previous
import functools

import jax
import jax.numpy as jnp
from jax import lax
from jax.experimental import pallas as pl
from jax.experimental.pallas import tpu as pltpu
from jax.experimental.pallas import tpu_sc as plsc

D = 32
DI = 16
CHUNK = 128
NW = 32
BIG = 1.0e30


def _take16(v, j):
    idx = jnp.full((16, 1), j, dtype=jnp.int32)
    dnums = lax.GatherDimensionNumbers(
        offset_dims=(), collapsed_slice_dims=(0,), start_index_map=(0,))
    return lax.gather(v, idx, dnums, (1,),
                      mode=lax.GatherScatterMode.PROMISE_IN_BOUNDS)


def _init_body(k, featp_ref, q0_ref, un_ref):
    x = featp_ref[...]
    lane = lax.broadcasted_iota(jnp.int32, x.shape, 1)
    xm = jnp.where(lane < k, x, -BIG)
    m = jnp.max(xm, axis=1, keepdims=True)
    e = jnp.exp(xm - m)
    s = jnp.sum(e, axis=1, keepdims=True)
    q0_ref[...] = e / s
    un_ref[...] = jnp.where(lane < k, m + jnp.log(s) - xm, BIG)


def _wvec(k, shape):
    lane = lax.broadcasted_iota(jnp.int32, shape, 1)
    return jnp.where(lane == k - 1, 10.0,
                     jnp.where(lane < k, 1.0, 0.0)).astype(jnp.float32)


def _update_body(k, log_out, agg_ref, un_ref, out_ref):
    agg = agg_ref[0] + agg_ref[1]
    w = _wvec(k, agg.shape)
    s = jnp.sum(agg * w, axis=1, keepdims=True)
    logit = w * w * agg - w * s - un_ref[...]
    m = jnp.max(logit, axis=1, keepdims=True)
    e = jnp.exp(logit - m)
    z = jnp.sum(e, axis=1, keepdims=True)
    if log_out:
        out_ref[...] = logit - m - jnp.log(z)
    else:
        out_ref[...] = e / z


def _edge_coef_kernel(n, e_total):
    ew = e_total // NW
    nchunks = ew // CHUNK
    mesh = plsc.VectorSubcoreMesh(core_axis_name="c", subcore_axis_name="s", num_cores=2, num_subcores=16)

    @functools.partial(
        pl.kernel,
        out_type=jax.ShapeDtypeStruct((e_total,), jnp.float32),
        mesh=mesh,
        compiler_params=pltpu.CompilerParams(use_tc_tiling_on_sc=False),
        scratch_types=[
            [pltpu.VMEM((CHUNK,), jnp.int32)] * 2,
            [pltpu.VMEM((CHUNK,), jnp.int32)] * 2,
            [pltpu.VMEM((CHUNK,), jnp.float32)] * 2,
            [pltpu.VMEM((CHUNK,), jnp.float32)] * 2,
            [[pltpu.VMEM((CHUNK,), jnp.float32)] * 3] * 2,
            [[pltpu.VMEM((CHUNK,), jnp.float32)] * 3] * 2,
            pltpu.VMEM((CHUNK,), jnp.float32),
            pltpu.VMEM((16,), jnp.float32),
            [pltpu.SemaphoreType.DMA] * 2,
            [pltpu.SemaphoreType.DMA] * 2,
        ],
    )
    def coef_kernel(ch0, ch1, ch2, ein, eout, d0, d1, params, ke_out,
                    einv, eoutv, d0v, d1v, rin, rout, kev, prmv, semi, semg):
        wid = lax.axis_index("c") * 16 + lax.axis_index("s")
        base0 = wid * ew
        pltpu.sync_copy(params, prmv)
        praw = prmv[...]
        pos = lax.iota(jnp.int32, 16)
        cv = jnp.where(pos < 7, 1.0 / (2.0 * praw * praw), jnp.abs(praw))
        a0 = _take16(cv, 0)
        a1 = _take16(cv, 1)
        b = [_take16(cv, 2), _take16(cv, 3), _take16(cv, 4)]
        g0 = _take16(cv, 5)
        g1 = _take16(cv, 6)
        aw1 = _take16(cv, 7)
        aw2 = _take16(cv, 8)
        chans = (ch0, ch1, ch2)

        def fire_idx(i, p):
            base = base0 + i * CHUNK
            pltpu.async_copy(ein.at[pl.ds(base, CHUNK)], einv[p], semi[p])
            pltpu.async_copy(eout.at[pl.ds(base, CHUNK)], eoutv[p], semi[p])
            pltpu.async_copy(d0.at[pl.ds(base, CHUNK)], d0v[p], semi[p])
            pltpu.async_copy(d1.at[pl.ds(base, CHUNK)], d1v[p], semi[p])

        def wait_idx(p):
            for dst in (einv[p], eoutv[p], d0v[p], d1v[p]):
                pltpu.make_async_copy(d0.at[pl.ds(0, CHUNK)], dst, semi[p]).wait()

        def fire_gather(p):
            for ch in range(3):
                pltpu.async_copy(chans[ch].at[einv[p]], rin[p][ch], semg[p])
                pltpu.async_copy(chans[ch].at[eoutv[p]], rout[p][ch], semg[p])

        def wait_gather(p):
            for ch in range(3):
                pltpu.make_async_copy(ch0.at[einv[p]], rin[p][ch], semg[p]).wait()
                pltpu.make_async_copy(ch0.at[einv[p]], rout[p][ch], semg[p]).wait()

        fire_idx(0, 0)
        fire_idx(1, 1)
        wait_idx(0)
        fire_gather(0)

        def step(i, par):
            @pl.when(i + 1 < nchunks)
            def _():
                wait_idx(1 - par)
                fire_gather(1 - par)
            wait_gather(par)
            for gidx in range(CHUNK // 16):
                sl = pl.ds(gidx * 16, 16)
                acc = jnp.zeros((16,), jnp.float32)
                for ch in range(3):
                    dd = rin[par][ch][sl] - rout[par][ch][sl]
                    acc = acc + dd * dd * b[ch]
                dv0 = d0v[par][sl]
                dv1 = d1v[par][sl]
                k1 = aw1 * jnp.exp(-(dv0 * a0 + dv1 * a1) - acc)
                k2 = aw2 * jnp.exp(-(dv0 * g0 + dv1 * g1))
                kev[sl] = k1 + k2
            base = base0 + i * CHUNK
            pltpu.sync_copy(kev, ke_out.at[pl.ds(base, CHUNK)])

            @pl.when(i + 2 < nchunks)
            def _():
                fire_idx(i + 2, par)

        def body2(i2, _):
            step(i2 * 2, 0)
            step(i2 * 2 + 1, 1)
            return _

        lax.fori_loop(0, nchunks // 2, body2, None)

    return coef_kernel


def _agg_kernel(n, e_total):
    ew = e_total // NW
    ca = 2 * CHUNK
    nchunks = ew // ca
    rows_per_tile = n // 16
    zc = rows_per_tile // 14
    mesh = plsc.VectorSubcoreMesh(core_axis_name="c", subcore_axis_name="s", num_cores=2, num_subcores=16)

    @functools.partial(
        pl.kernel,
        out_type=jax.ShapeDtypeStruct((2, n, D), jnp.float32),
        mesh=mesh,
        compiler_params=pltpu.CompilerParams(use_tc_tiling_on_sc=False),
        scratch_types=[
            pltpu.VMEM_SHARED((n, D), jnp.float32),
            [pltpu.VMEM((2, CHUNK), jnp.int32)] * 2,
            [pltpu.VMEM((2, CHUNK), jnp.int32)] * 2,
            [pltpu.VMEM((ca,), jnp.float32)] * 2,
            [pltpu.VMEM((ca, D), jnp.float32)] * 2,
            [pltpu.SemaphoreType.DMA] * 2,
            [pltpu.SemaphoreType.DMA] * 2,
        ],
    )
    def agg_kernel(qp, ein2, eout2, ke, agg_out,
                   aggs, einv, eoutv, kev, rows, semi, semg):
        cid = lax.axis_index("c")
        sid = lax.axis_index("s")
        wid = cid * 16 + sid
        base0 = wid * ew
        r0 = sid * rows_per_tile

        zb = rows[0]
        def zrow(i, _):
            zb[i, pl.ds(0, 16)] = jnp.zeros((16,), jnp.float32)
            zb[i, pl.ds(16, 16)] = jnp.zeros((16,), jnp.float32)
            return _
        lax.fori_loop(0, zc, zrow, None)

        def zcopy(i, _):
            pltpu.sync_copy(zb.at[pl.ds(0, zc)], aggs.at[pl.ds(r0 + i * zc, zc)])
            return _
        lax.fori_loop(0, 14, zcopy, None)
        plsc.subcore_barrier()

        def fire_idx(i, p):
            base = base0 + i * ca
            rb = base // CHUNK
            pltpu.async_copy(ein2.at[pl.ds(rb, 2)], einv[p], semi[p])
            pltpu.async_copy(eout2.at[pl.ds(rb, 2)], eoutv[p], semi[p])
            pltpu.async_copy(ke.at[pl.ds(base, ca)], kev[p], semi[p])

        def wait_idx(p):
            pltpu.make_async_copy(ein2.at[pl.ds(0, 2)], einv[p], semi[p]).wait()
            pltpu.make_async_copy(ein2.at[pl.ds(0, 2)], eoutv[p], semi[p]).wait()
            pltpu.make_async_copy(ke.at[pl.ds(0, ca)], kev[p], semi[p]).wait()

        def fire_gather(p):
            for j in range(2):
                pltpu.async_copy(qp.at[einv[p].at[j]],
                                 rows[p].at[pl.ds(j * CHUNK, CHUNK)], semg[p])

        def wait_gather(p):
            for j in range(2):
                pltpu.make_async_copy(qp.at[einv[p].at[j]],
                                      rows[p].at[pl.ds(j * CHUNK, CHUNK)],
                                      semg[p]).wait()

        fire_idx(0, 0)
        fire_idx(1, 1)
        wait_idx(0)
        fire_gather(0)

        def step(i, par):
            @pl.when(i + 1 < nchunks)
            def _():
                wait_idx(1 - par)
                fire_gather(1 - par)
            wait_gather(par)
            rw = rows[par]
            kw = kev[par]
            for gidx in range(ca // 16):
                kv = kw[pl.ds(gidx * 16, 16)]
                for j in range(16):
                    e = gidx * 16 + j
                    s = _take16(kv, j)
                    rw[e, pl.ds(0, 16)] = rw[e, pl.ds(0, 16)] * s
                    rw[e, pl.ds(16, 16)] = rw[e, pl.ds(16, 16)] * s
            for j in range(2):
                pltpu.sync_copy(rw.at[pl.ds(j * CHUNK, CHUNK)],
                                aggs.at[eoutv[par].at[j]], add=True)

            @pl.when(i + 2 < nchunks)
            def _():
                fire_idx(i + 2, par)

        def body2(i2, _):
            step(i2 * 2, 0)
            step(i2 * 2 + 1, 1)
            return _

        lax.fori_loop(0, nchunks // 2, body2, None)
        plsc.subcore_barrier()

        def ocopy(i, _):
            pltpu.sync_copy(aggs.at[pl.ds(r0 + i * zc, zc)],
                            agg_out.at[cid, pl.ds(r0 + i * zc, zc)])
            return _
        lax.fori_loop(0, 14, ocopy, None)

    return agg_kernel


def kernel(feat, img, dist_diff, msg_node, alpha, beta, gamma, w1, w2):
    B, K, H, W = feat.shape
    N = B * H * W
    E = msg_node.shape[0]
    MAX_IT = 10

    featp = jnp.pad(feat.reshape(K, N).T, ((0, 0), (0, D - K)))
    imgc = img.reshape(3, N)
    ein = msg_node[:, 0]
    eout = msg_node[:, 1]
    d0 = dist_diff[:, 0]
    d1 = dist_diff[:, 1]
    params = jnp.concatenate([alpha.ravel(), beta.ravel(), gamma.ravel(),
                              w1.ravel(), w2.ravel(),
                              jnp.ones((7,), jnp.float32)])

    RB = 256
    grid = (N // RB,)

    q0, unary = pl.pallas_call(
        functools.partial(_init_body, K),
        grid=grid,
        in_specs=[pl.BlockSpec((RB, D), lambda i: (i, 0))],
        out_specs=[pl.BlockSpec((RB, D), lambda i: (i, 0))] * 2,
        out_shape=[jax.ShapeDtypeStruct((N, D), jnp.float32)] * 2,
    )(featp)

    ke = _edge_coef_kernel(N, E)(imgc[0], imgc[1], imgc[2],
                                 ein, eout, d0, d1, params)

    agg_k = _agg_kernel(N, E)
    ein2 = ein.reshape(E // CHUNK, CHUNK)
    eout2 = eout.reshape(E // CHUNK, CHUNK)

    def update(log_out, agg2, un):
        return pl.pallas_call(
            functools.partial(_update_body, K, log_out),
            grid=grid,
            in_specs=[pl.BlockSpec((2, RB, D), lambda i: (0, i, 0)),
                      pl.BlockSpec((RB, D), lambda i: (i, 0))],
            out_specs=pl.BlockSpec((RB, D), lambda i: (i, 0)),
            out_shape=jax.ShapeDtypeStruct((N, D), jnp.float32),
        )(agg2, un)

    q = q0
    for it in range(MAX_IT):
        agg2 = agg_k(q, ein2, eout2, ke)
        q = update(it == MAX_IT - 1, agg2, unary)

    logq = q[:, :K].reshape(B, H, W, K)
    return jnp.transpose(logq, (0, 3, 1, 2))


# speedup vs baseline: 20.2494x; 1.3663x over previous
import functools

import jax
import jax.numpy as jnp
from jax import lax
from jax.experimental import pallas as pl
from jax.experimental.pallas import tpu as pltpu
from jax.experimental.pallas import tpu_sc as plsc

D = 32
DI = 16
CHUNK = 128
NW = 32
BIG = 1.0e30


def _take16(v, j):
    idx = jnp.full((16, 1), j, dtype=jnp.int32)
    dnums = lax.GatherDimensionNumbers(
        offset_dims=(), collapsed_slice_dims=(0,), start_index_map=(0,))
    return lax.gather(v, idx, dnums, (1,),
                      mode=lax.GatherScatterMode.PROMISE_IN_BOUNDS)


def _init_body(k, featp_ref, q0_ref, un_ref):
    x = featp_ref[...]
    lane = lax.broadcasted_iota(jnp.int32, x.shape, 1)
    xm = jnp.where(lane < k, x, -BIG)
    m = jnp.max(xm, axis=1, keepdims=True)
    e = jnp.exp(xm - m)
    s = jnp.sum(e, axis=1, keepdims=True)
    q0_ref[...] = e / s
    un_ref[...] = jnp.where(lane < k, m + jnp.log(s) - xm, BIG)


def _wvec(k, shape):
    lane = lax.broadcasted_iota(jnp.int32, shape, 1)
    return jnp.where(lane == k - 1, 10.0,
                     jnp.where(lane < k, 1.0, 0.0)).astype(jnp.float32)


def _update_body(k, log_out, agg_ref, un_ref, out_ref):
    agg = agg_ref[0] + agg_ref[1]
    w = _wvec(k, agg.shape)
    s = jnp.sum(agg * w, axis=1, keepdims=True)
    logit = w * w * agg - w * s - un_ref[...]
    m = jnp.max(logit, axis=1, keepdims=True)
    e = jnp.exp(logit - m)
    z = jnp.sum(e, axis=1, keepdims=True)
    if log_out:
        out_ref[...] = logit - m - jnp.log(z)
    else:
        out_ref[...] = e / z


def _edge_coef_kernel(n, e_total):
    ew = e_total // NW
    nchunks = ew // CHUNK
    mesh = plsc.VectorSubcoreMesh(core_axis_name="c", subcore_axis_name="s", num_cores=2, num_subcores=16)

    @functools.partial(
        pl.kernel,
        out_type=jax.ShapeDtypeStruct((e_total,), jnp.float32),
        mesh=mesh,
        compiler_params=pltpu.CompilerParams(use_tc_tiling_on_sc=False),
        scratch_types=[
            [pltpu.VMEM((CHUNK,), jnp.int32)] * 2,
            [pltpu.VMEM((CHUNK,), jnp.int32)] * 2,
            [pltpu.VMEM((CHUNK,), jnp.float32)] * 2,
            [pltpu.VMEM((CHUNK,), jnp.float32)] * 2,
            [[pltpu.VMEM((CHUNK,), jnp.float32)] * 3] * 2,
            [[pltpu.VMEM((CHUNK,), jnp.float32)] * 3] * 2,
            pltpu.VMEM((CHUNK,), jnp.float32),
            pltpu.VMEM((16,), jnp.float32),
            [pltpu.SemaphoreType.DMA] * 2,
            [pltpu.SemaphoreType.DMA] * 2,
        ],
    )
    def coef_kernel(ch0, ch1, ch2, ein, eout, d0, d1, params, ke_out,
                    einv, eoutv, d0v, d1v, rin, rout, kev, prmv, semi, semg):
        wid = lax.axis_index("c") * 16 + lax.axis_index("s")
        base0 = wid * ew
        pltpu.sync_copy(params, prmv)
        praw = prmv[...]
        pos = lax.iota(jnp.int32, 16)
        cv = jnp.where(pos < 7, 1.0 / (2.0 * praw * praw), jnp.abs(praw))
        a0 = _take16(cv, 0)
        a1 = _take16(cv, 1)
        b = [_take16(cv, 2), _take16(cv, 3), _take16(cv, 4)]
        g0 = _take16(cv, 5)
        g1 = _take16(cv, 6)
        aw1 = _take16(cv, 7)
        aw2 = _take16(cv, 8)
        chans = (ch0, ch1, ch2)

        def fire_idx(i, p):
            base = base0 + i * CHUNK
            pltpu.async_copy(ein.at[pl.ds(base, CHUNK)], einv[p], semi[p])
            pltpu.async_copy(eout.at[pl.ds(base, CHUNK)], eoutv[p], semi[p])
            pltpu.async_copy(d0.at[pl.ds(base, CHUNK)], d0v[p], semi[p])
            pltpu.async_copy(d1.at[pl.ds(base, CHUNK)], d1v[p], semi[p])

        def wait_idx(p):
            for dst in (einv[p], eoutv[p], d0v[p], d1v[p]):
                pltpu.make_async_copy(d0.at[pl.ds(0, CHUNK)], dst, semi[p]).wait()

        def fire_gather(p):
            for ch in range(3):
                pltpu.async_copy(chans[ch].at[einv[p]], rin[p][ch], semg[p])
                pltpu.async_copy(chans[ch].at[eoutv[p]], rout[p][ch], semg[p])

        def wait_gather(p):
            for ch in range(3):
                pltpu.make_async_copy(ch0.at[einv[p]], rin[p][ch], semg[p]).wait()
                pltpu.make_async_copy(ch0.at[einv[p]], rout[p][ch], semg[p]).wait()

        fire_idx(0, 0)
        fire_idx(1, 1)
        wait_idx(0)
        fire_gather(0)

        def step(i, par):
            @pl.when(i + 1 < nchunks)
            def _():
                wait_idx(1 - par)
                fire_gather(1 - par)
            wait_gather(par)
            for gidx in range(CHUNK // 16):
                sl = pl.ds(gidx * 16, 16)
                acc = jnp.zeros((16,), jnp.float32)
                for ch in range(3):
                    dd = rin[par][ch][sl] - rout[par][ch][sl]
                    acc = acc + dd * dd * b[ch]
                dv0 = d0v[par][sl]
                dv1 = d1v[par][sl]
                k1 = aw1 * jnp.exp(-(dv0 * a0 + dv1 * a1) - acc)
                k2 = aw2 * jnp.exp(-(dv0 * g0 + dv1 * g1))
                kev[sl] = k1 + k2
            base = base0 + i * CHUNK
            pltpu.sync_copy(kev, ke_out.at[pl.ds(base, CHUNK)])

            @pl.when(i + 2 < nchunks)
            def _():
                fire_idx(i + 2, par)

        def body2(i2, _):
            step(i2 * 2, 0)
            step(i2 * 2 + 1, 1)
            return _

        lax.fori_loop(0, nchunks // 2, body2, None)

    return coef_kernel


def _agg_kernel(n, e_total):
    ew = e_total // NW
    ca = 2 * CHUNK
    nchunks = ew // ca
    rows_per_tile = n // 16
    zc = rows_per_tile // 14
    mesh = plsc.VectorSubcoreMesh(core_axis_name="c", subcore_axis_name="s", num_cores=2, num_subcores=16)

    @functools.partial(
        pl.kernel,
        out_type=jax.ShapeDtypeStruct((2, n, D), jnp.float32),
        mesh=mesh,
        compiler_params=pltpu.CompilerParams(use_tc_tiling_on_sc=False),
        scratch_types=[
            pltpu.VMEM_SHARED((n, D), jnp.float32),
            [pltpu.VMEM((2, CHUNK), jnp.int32)] * 2,
            [pltpu.VMEM((2, CHUNK), jnp.int32)] * 2,
            [pltpu.VMEM((ca,), jnp.float32)] * 2,
            [pltpu.VMEM((ca, D), jnp.float32)] * 2,
            [pltpu.SemaphoreType.DMA] * 2,
            [pltpu.SemaphoreType.DMA] * 2,
        ],
    )
    def agg_kernel(qp, ein2, eout2, ke, agg_out,
                   aggs, einv, eoutv, kev, rows, semi, semg):
        cid = lax.axis_index("c")
        sid = lax.axis_index("s")
        wid = cid * 16 + sid
        base0 = wid * ew
        r0 = sid * rows_per_tile

        zb = rows[0]
        def zrow(i, _):
            zb[i, pl.ds(0, 16)] = jnp.zeros((16,), jnp.float32)
            zb[i, pl.ds(16, 16)] = jnp.zeros((16,), jnp.float32)
            return _
        lax.fori_loop(0, zc, zrow, None)

        def zcopy(i, _):
            pltpu.sync_copy(zb.at[pl.ds(0, zc)], aggs.at[pl.ds(r0 + i * zc, zc)])
            return _
        lax.fori_loop(0, 14, zcopy, None)
        plsc.subcore_barrier()

        def fire_idx(i, p):
            base = base0 + i * ca
            rb = base // CHUNK
            pltpu.async_copy(ein2.at[pl.ds(rb, 2)], einv[p], semi[p])
            pltpu.async_copy(eout2.at[pl.ds(rb, 2)], eoutv[p], semi[p])
            pltpu.async_copy(ke.at[pl.ds(base, ca)], kev[p], semi[p])

        def wait_idx(p):
            pltpu.make_async_copy(ein2.at[pl.ds(0, 2)], einv[p], semi[p]).wait()
            pltpu.make_async_copy(ein2.at[pl.ds(0, 2)], eoutv[p], semi[p]).wait()
            pltpu.make_async_copy(ke.at[pl.ds(0, ca)], kev[p], semi[p]).wait()

        def fire_gather(p):
            for j in range(2):
                pltpu.async_copy(qp.at[einv[p].at[j]],
                                 rows[p].at[pl.ds(j * CHUNK, CHUNK)], semg[p])

        def wait_gather(p):
            for j in range(2):
                pltpu.make_async_copy(qp.at[einv[p].at[j]],
                                      rows[p].at[pl.ds(j * CHUNK, CHUNK)],
                                      semg[p]).wait()

        fire_idx(0, 0)
        fire_idx(1, 1)
        wait_idx(0)
        fire_gather(0)

        def step(i, par):
            @pl.when(i + 1 < nchunks)
            def _():
                wait_idx(1 - par)
                fire_gather(1 - par)
            wait_gather(par)
            rw = rows[par]
            kw = kev[par]
            for gidx in range(ca // 16):
                kv = kw[pl.ds(gidx * 16, 16)]
                for j in range(16):
                    e = gidx * 16 + j
                    s = _take16(kv, j)
                    rw[e, pl.ds(0, 16)] = rw[e, pl.ds(0, 16)] * s
                    rw[e, pl.ds(16, 16)] = rw[e, pl.ds(16, 16)] * s
            for j in range(2):
                pltpu.sync_copy(rw.at[pl.ds(j * CHUNK, CHUNK)],
                                aggs.at[eoutv[par].at[j]], add=True)

            @pl.when(i + 2 < nchunks)
            def _():
                fire_idx(i + 2, par)

        def body2(i2, _):
            step(i2 * 2, 0)
            step(i2 * 2 + 1, 1)
            return _

        lax.fori_loop(0, nchunks // 2, body2, None)
        plsc.subcore_barrier()

        def ocopy(i, _):
            pltpu.sync_copy(aggs.at[pl.ds(r0 + i * zc, zc)],
                            agg_out.at[cid, pl.ds(r0 + i * zc, zc)])
            return _
        lax.fori_loop(0, 14, ocopy, None)

    return agg_kernel


def kernel(feat, img, dist_diff, msg_node, alpha, beta, gamma, w1, w2):
    B, K, H, W = feat.shape
    N = B * H * W
    E = msg_node.shape[0]
    MAX_IT = 10

    featp = jnp.pad(feat.reshape(K, N).T, ((0, 0), (0, D - K)))
    imgc = img.reshape(3, N)
    ein = msg_node[:, 0]
    eout = msg_node[:, 1]
    d0 = dist_diff[:, 0]
    d1 = dist_diff[:, 1]
    params = jnp.concatenate([alpha.ravel(), beta.ravel(), gamma.ravel(),
                              w1.ravel(), w2.ravel(),
                              jnp.ones((7,), jnp.float32)])

    RB = 3584
    grid = (N // RB,)

    q0, unary = pl.pallas_call(
        functools.partial(_init_body, K),
        grid=grid,
        in_specs=[pl.BlockSpec((RB, D), lambda i: (i, 0))],
        out_specs=[pl.BlockSpec((RB, D), lambda i: (i, 0))] * 2,
        out_shape=[jax.ShapeDtypeStruct((N, D), jnp.float32)] * 2,
    )(featp)

    ke = _edge_coef_kernel(N, E)(imgc[0], imgc[1], imgc[2],
                                 ein, eout, d0, d1, params)

    agg_k = _agg_kernel(N, E)
    ein2 = ein.reshape(E // CHUNK, CHUNK)
    eout2 = eout.reshape(E // CHUNK, CHUNK)

    def update(log_out, agg2, un):
        return pl.pallas_call(
            functools.partial(_update_body, K, log_out),
            grid=grid,
            in_specs=[pl.BlockSpec((2, RB, D), lambda i: (0, i, 0)),
                      pl.BlockSpec((RB, D), lambda i: (i, 0))],
            out_specs=pl.BlockSpec((RB, D), lambda i: (i, 0)),
            out_shape=jax.ShapeDtypeStruct((N, D), jnp.float32),
        )(agg2, un)

    q = q0
    for it in range(MAX_IT):
        agg2 = agg_k(q, ein2, eout2, ke)
        q = update(it == MAX_IT - 1, agg2, unary)

    logq = q[:, :K].reshape(B, H, W, K)
    return jnp.transpose(logq, (0, 3, 1, 2))


# speedup vs baseline: 22.6278x; 1.1175x over previous
import functools

import jax
import jax.numpy as jnp
from jax import lax
from jax.experimental import pallas as pl
from jax.experimental.pallas import tpu as pltpu
from jax.experimental.pallas import tpu_sc as plsc

D = 32
DI = 16
CHUNK = 128
NW = 32
BIG = 1.0e30


def _take16(v, j):
    idx = jnp.full((16, 1), j, dtype=jnp.int32)
    dnums = lax.GatherDimensionNumbers(
        offset_dims=(), collapsed_slice_dims=(0,), start_index_map=(0,))
    return lax.gather(v, idx, dnums, (1,),
                      mode=lax.GatherScatterMode.PROMISE_IN_BOUNDS)


def _rot16(v, sh):
    idx = ((lax.iota(jnp.int32, 16) + sh) % 16).reshape(16, 1)
    dnums = lax.GatherDimensionNumbers(
        offset_dims=(), collapsed_slice_dims=(0,), start_index_map=(0,))
    return lax.gather(v, idx, dnums, (1,),
                      mode=lax.GatherScatterMode.PROMISE_IN_BOUNDS)


def _vsum16(v):
    for sh in (8, 4, 2, 1):
        v = v + _rot16(v, sh)
    return v


def _lane0(v):
    return lax.squeeze(lax.slice(v, (0,), (1,)), (0,))


def _init_body(k, featp_ref, q0_ref, un_ref):
    x = featp_ref[...]
    lane = lax.broadcasted_iota(jnp.int32, x.shape, 1)
    xm = jnp.where(lane < k, x, -BIG)
    m = jnp.max(xm, axis=1, keepdims=True)
    e = jnp.exp(xm - m)
    s = jnp.sum(e, axis=1, keepdims=True)
    q0_ref[...] = e / s
    un_ref[...] = jnp.where(lane < k, m + jnp.log(s) - xm, BIG)


def _wvec(k, shape):
    lane = lax.broadcasted_iota(jnp.int32, shape, 1)
    return jnp.where(lane == k - 1, 10.0,
                     jnp.where(lane < k, 1.0, 0.0)).astype(jnp.float32)


def _update_body(k, log_out, agg_ref, un_ref, out_ref):
    agg = agg_ref[0] + agg_ref[1]
    w = _wvec(k, agg.shape)
    s = jnp.sum(agg * w, axis=1, keepdims=True)
    logit = w * w * agg - w * s - un_ref[...]
    m = jnp.max(logit, axis=1, keepdims=True)
    e = jnp.exp(logit - m)
    z = jnp.sum(e, axis=1, keepdims=True)
    if log_out:
        out_ref[...] = logit - m - jnp.log(z)
    else:
        out_ref[...] = e / z


def _edge_coef_kernel(n, e_total):
    ew = e_total // NW
    nchunks = ew // CHUNK
    mesh = plsc.VectorSubcoreMesh(core_axis_name="c", subcore_axis_name="s", num_cores=2, num_subcores=16)

    @functools.partial(
        pl.kernel,
        out_type=jax.ShapeDtypeStruct((e_total,), jnp.float32),
        mesh=mesh,
        compiler_params=pltpu.CompilerParams(use_tc_tiling_on_sc=False),
        scratch_types=[
            [pltpu.VMEM((CHUNK,), jnp.int32)] * 2,
            [pltpu.VMEM((CHUNK,), jnp.int32)] * 2,
            [pltpu.VMEM((CHUNK,), jnp.float32)] * 2,
            [pltpu.VMEM((CHUNK,), jnp.float32)] * 2,
            [[pltpu.VMEM((CHUNK,), jnp.float32)] * 3] * 2,
            [[pltpu.VMEM((CHUNK,), jnp.float32)] * 3] * 2,
            pltpu.VMEM((CHUNK,), jnp.float32),
            pltpu.VMEM((16,), jnp.float32),
            [pltpu.SemaphoreType.DMA] * 2,
            [pltpu.SemaphoreType.DMA] * 2,
        ],
    )
    def coef_kernel(ch0, ch1, ch2, ein, eout, d0, d1, params, ke_out,
                    einv, eoutv, d0v, d1v, rin, rout, kev, prmv, semi, semg):
        wid = lax.axis_index("c") * 16 + lax.axis_index("s")
        base0 = wid * ew
        pltpu.sync_copy(params, prmv)
        praw = prmv[...]
        pos = lax.iota(jnp.int32, 16)
        cv = jnp.where(pos < 7, 1.0 / (2.0 * praw * praw), jnp.abs(praw))
        a0 = _take16(cv, 0)
        a1 = _take16(cv, 1)
        b = [_take16(cv, 2), _take16(cv, 3), _take16(cv, 4)]
        g0 = _take16(cv, 5)
        g1 = _take16(cv, 6)
        aw1 = _take16(cv, 7)
        aw2 = _take16(cv, 8)
        chans = (ch0, ch1, ch2)

        def fire_idx(i, p):
            base = base0 + i * CHUNK
            pltpu.async_copy(ein.at[pl.ds(base, CHUNK)], einv[p], semi[p])
            pltpu.async_copy(eout.at[pl.ds(base, CHUNK)], eoutv[p], semi[p])
            pltpu.async_copy(d0.at[pl.ds(base, CHUNK)], d0v[p], semi[p])
            pltpu.async_copy(d1.at[pl.ds(base, CHUNK)], d1v[p], semi[p])

        def wait_idx(p):
            for dst in (einv[p], eoutv[p], d0v[p], d1v[p]):
                pltpu.make_async_copy(d0.at[pl.ds(0, CHUNK)], dst, semi[p]).wait()

        def fire_gather(p):
            for ch in range(3):
                pltpu.async_copy(chans[ch].at[einv[p]], rin[p][ch], semg[p])
                pltpu.async_copy(chans[ch].at[eoutv[p]], rout[p][ch], semg[p])

        def wait_gather(p):
            for ch in range(3):
                pltpu.make_async_copy(ch0.at[einv[p]], rin[p][ch], semg[p]).wait()
                pltpu.make_async_copy(ch0.at[einv[p]], rout[p][ch], semg[p]).wait()

        fire_idx(0, 0)
        fire_idx(1, 1)
        wait_idx(0)
        fire_gather(0)

        def step(i, par):
            @pl.when(i + 1 < nchunks)
            def _():
                wait_idx(1 - par)
                fire_gather(1 - par)
            wait_gather(par)
            for gidx in range(CHUNK // 16):
                sl = pl.ds(gidx * 16, 16)
                acc = jnp.zeros((16,), jnp.float32)
                for ch in range(3):
                    dd = rin[par][ch][sl] - rout[par][ch][sl]
                    acc = acc + dd * dd * b[ch]
                dv0 = d0v[par][sl]
                dv1 = d1v[par][sl]
                k1 = aw1 * jnp.exp(-(dv0 * a0 + dv1 * a1) - acc)
                k2 = aw2 * jnp.exp(-(dv0 * g0 + dv1 * g1))
                kev[sl] = k1 + k2
            base = base0 + i * CHUNK
            pltpu.sync_copy(kev, ke_out.at[pl.ds(base, CHUNK)])

            @pl.when(i + 2 < nchunks)
            def _():
                fire_idx(i + 2, par)

        def body2(i2, _):
            step(i2 * 2, 0)
            step(i2 * 2 + 1, 1)
            return _

        lax.fori_loop(0, nchunks // 2, body2, None)

    return coef_kernel


def _fused_kernel(n, e_total, max_it):
    ew = e_total // NW
    ca = 2 * CHUNK
    nchunks = ew // ca
    rpt = n // 16
    half = n // 2
    rhalf = n // 32
    zc = rpt // 14
    sm = 112
    nsm = rhalf // sm
    mesh = plsc.VectorSubcoreMesh(core_axis_name="c", subcore_axis_name="s", num_cores=2, num_subcores=16)

    @functools.partial(
        pl.kernel,
        out_type=[jax.ShapeDtypeStruct((2, n, D), jnp.float32),
                  jax.ShapeDtypeStruct((n, D), jnp.float32),
                  jax.ShapeDtypeStruct((2, half, D), jnp.float32),
                  jax.ShapeDtypeStruct((2, 20, 16), jnp.float32)],
        mesh=mesh,
        compiler_params=pltpu.CompilerParams(use_tc_tiling_on_sc=False),
        scratch_types=[
            pltpu.VMEM_SHARED((n, D), jnp.float32),
            [pltpu.VMEM((2, CHUNK), jnp.int32)] * 2,
            [pltpu.VMEM((2, CHUNK), jnp.int32)] * 2,
            [pltpu.VMEM((ca,), jnp.float32)] * 2,
            [pltpu.VMEM((ca, D), jnp.float32)] * 2,
            pltpu.VMEM((16,), jnp.float32),
            pltpu.VMEM((16,), jnp.float32),
            pltpu.VMEM((20, 16), jnp.float32),
            [pltpu.SemaphoreType.DMA] * 2,
            [pltpu.SemaphoreType.DMA] * 2,
        ],
    )
    def fused_kernel(q0, ein2, eout2, ke, unary,
                     agg_out, qcur, xbuf, flags,
                     aggs, einv, eoutv, kev, rows, fones, fbuf, zf, semi, semg):
        cid = lax.axis_index("c")
        sid = lax.axis_index("s")
        wid = cid * 16 + sid
        base0 = wid * ew
        r0 = sid * rpt
        oth = 1 - cid

        fones[...] = jnp.ones((16,), jnp.float32)

        @pl.when(sid == 0)
        def _():
            def zfrow(i, _):
                zf[i, pl.ds(0, 16)] = jnp.zeros((16,), jnp.float32)
                return _
            lax.fori_loop(0, 20, zfrow, None)
            pltpu.sync_copy(zf, flags.at[cid])

        def wflag(ev):
            @pl.when(sid == 0)
            def _():
                pltpu.sync_copy(fones, flags.at[cid, ev])

        def poll(ev):
            fbuf[...] = jnp.zeros((16,), jnp.float32)

            def pbody(k, done):
                @pl.when(done < 0.5)
                def _():
                    pltpu.sync_copy(flags.at[oth, ev], fbuf)
                return jnp.maximum(done, _lane0(fbuf[...]))

            d = lax.fori_loop(0, 256, pbody, jnp.float32(0))

            @pl.when(d < 0.5)
            def _():
                lax.fori_loop(0, 8192, pbody, jnp.float32(0))

        def fill_zeros():
            def zrow(i, _):
                rows[0][i, pl.ds(0, 16)] = jnp.zeros((16,), jnp.float32)
                rows[0][i, pl.ds(16, 16)] = jnp.zeros((16,), jnp.float32)
                return _
            lax.fori_loop(0, zc, zrow, None)

        def zero_aggs():
            def zcopy(i, _):
                pltpu.sync_copy(rows[0].at[pl.ds(0, zc)],
                                aggs.at[pl.ds(r0 + i * zc, zc)])
                return _
            lax.fori_loop(0, 14, zcopy, None)

        fill_zeros()
        zero_aggs()

        def icopy(i, _):
            pltpu.sync_copy(q0.at[pl.ds(r0 + i * zc, zc)], rows[1].at[pl.ds(0, zc)])
            pltpu.sync_copy(rows[1].at[pl.ds(0, zc)], qcur.at[pl.ds(r0 + i * zc, zc)])
            return _
        lax.fori_loop(0, 14, icopy, None)
        plsc.subcore_barrier()
        wflag(0)
        poll(0)

        def fire_idx(i, p):
            base = base0 + i * ca
            rb = base // CHUNK
            pltpu.async_copy(ein2.at[pl.ds(rb, 2)], einv[p], semi[p])
            pltpu.async_copy(eout2.at[pl.ds(rb, 2)], eoutv[p], semi[p])
            pltpu.async_copy(ke.at[pl.ds(base, ca)], kev[p], semi[p])

        def wait_idx(p):
            pltpu.make_async_copy(ein2.at[pl.ds(0, 2)], einv[p], semi[p]).wait()
            pltpu.make_async_copy(ein2.at[pl.ds(0, 2)], eoutv[p], semi[p]).wait()
            pltpu.make_async_copy(ke.at[pl.ds(0, ca)], kev[p], semi[p]).wait()

        def fire_gather(p):
            for j in range(2):
                pltpu.async_copy(qcur.at[einv[p].at[j]],
                                 rows[p].at[pl.ds(j * CHUNK, CHUNK)], semg[p])

        def wait_gather(p):
            for j in range(2):
                pltpu.make_async_copy(qcur.at[einv[p].at[j]],
                                      rows[p].at[pl.ds(j * CHUNK, CHUNK)],
                                      semg[p]).wait()

        def edge_phase():
            fire_idx(0, 0)
            fire_idx(1, 1)
            wait_idx(0)
            fire_gather(0)

            def step(i, par):
                @pl.when(i + 1 < nchunks)
                def _():
                    wait_idx(1 - par)
                    fire_gather(1 - par)
                wait_gather(par)
                rw = rows[par]
                kw = kev[par]

                def gloop(g, _):
                    kv = kw[pl.ds(g * 16, 16)]
                    for j in range(16):
                        e = g * 16 + j
                        s = _take16(kv, j)
                        rw[e, pl.ds(0, 16)] = rw[e, pl.ds(0, 16)] * s
                        rw[e, pl.ds(16, 16)] = rw[e, pl.ds(16, 16)] * s
                    return _
                lax.fori_loop(0, ca // 16, gloop, None)
                for j in range(2):
                    pltpu.sync_copy(rw.at[pl.ds(j * CHUNK, CHUNK)],
                                    aggs.at[eoutv[par].at[j]], add=True)

                @pl.when(i + 2 < nchunks)
                def _():
                    fire_idx(i + 2, par)

            def body2(i2, _):
                step(i2 * 2, 0)
                step(i2 * 2 + 1, 1)
                return _
            lax.fori_loop(0, nchunks // 2, body2, None)

        w1v = jnp.where(lax.iota(jnp.int32, 16) == 4, 10.0,
                        jnp.where(lax.iota(jnp.int32, 16) < 5, 1.0, 0.0)
                        ).astype(jnp.float32)

        def sm_phase():
            def smchunk(i, _):
                rb = cid * half + sid * rhalf + i * sm
                rel = sid * rhalf + i * sm
                pltpu.sync_copy(aggs.at[pl.ds(rb, sm)],
                                rows[0].at[pl.ds(0, sm)])
                pltpu.sync_copy(xbuf.at[oth, pl.ds(rel, sm)],
                                rows[0].at[pl.ds(sm, sm)])
                pltpu.sync_copy(unary.at[pl.ds(rb, sm)],
                                rows[1].at[pl.ds(0, sm)])

                def node(r, _):
                    rx = sm + r
                    a0 = rows[0][r, pl.ds(0, 16)] + rows[0][rx, pl.ds(0, 16)]
                    a1 = rows[0][r, pl.ds(16, 16)] + rows[0][rx, pl.ds(16, 16)]
                    u0 = rows[1][r, pl.ds(0, 16)]
                    u1 = rows[1][r, pl.ds(16, 16)]
                    s = _vsum16(a0 + a1 * w1v)
                    l0 = a0 - s - u0
                    l1 = w1v * w1v * a1 - w1v * s - u1
                    e0 = jnp.exp(l0)
                    e1 = jnp.exp(l1)
                    rz = 1.0 / _vsum16(e0 + e1)
                    rows[1][rx, pl.ds(0, 16)] = e0 * rz
                    rows[1][rx, pl.ds(16, 16)] = e1 * rz
                    return _
                lax.fori_loop(0, sm, node, None)
                pltpu.sync_copy(rows[1].at[pl.ds(sm, sm)],
                                qcur.at[pl.ds(rb, sm)])
                return _
            lax.fori_loop(0, nsm, smchunk, None)

        for t in range(max_it):
            if t > 0:
                poll(2 * t)
            edge_phase()
            plsc.subcore_barrier()

            if t < max_it - 1:
                in_oth = (r0 >= oth * half) & (r0 < (oth + 1) * half)

                @pl.when(in_oth)
                def _():
                    def xcopy(i, _):
                        pltpu.sync_copy(
                            aggs.at[pl.ds(r0 + i * zc, zc)],
                            xbuf.at[cid, pl.ds(r0 - oth * half + i * zc, zc)])
                        return _
                    lax.fori_loop(0, 14, xcopy, None)
                plsc.subcore_barrier()
                wflag(1 + 2 * t)
                poll(1 + 2 * t)
                sm_phase()
                plsc.subcore_barrier()
                wflag(2 + 2 * t)
                fill_zeros()
                zero_aggs()
                plsc.subcore_barrier()
            else:
                def ocopy(i, _):
                    pltpu.sync_copy(aggs.at[pl.ds(r0 + i * zc, zc)],
                                    agg_out.at[cid, pl.ds(r0 + i * zc, zc)])
                    return _
                lax.fori_loop(0, 14, ocopy, None)

    return fused_kernel


def _agg_kernel(n, e_total):
    ew = e_total // NW
    ca = 2 * CHUNK
    nchunks = ew // ca
    rows_per_tile = n // 16
    zc = rows_per_tile // 14
    mesh = plsc.VectorSubcoreMesh(core_axis_name="c", subcore_axis_name="s", num_cores=2, num_subcores=16)

    @functools.partial(
        pl.kernel,
        out_type=jax.ShapeDtypeStruct((2, n, D), jnp.float32),
        mesh=mesh,
        compiler_params=pltpu.CompilerParams(use_tc_tiling_on_sc=False),
        scratch_types=[
            pltpu.VMEM_SHARED((n, D), jnp.float32),
            [pltpu.VMEM((2, CHUNK), jnp.int32)] * 2,
            [pltpu.VMEM((2, CHUNK), jnp.int32)] * 2,
            [pltpu.VMEM((ca,), jnp.float32)] * 2,
            [pltpu.VMEM((ca, D), jnp.float32)] * 2,
            [pltpu.SemaphoreType.DMA] * 2,
            [pltpu.SemaphoreType.DMA] * 2,
        ],
    )
    def agg_kernel(qp, ein2, eout2, ke, agg_out,
                   aggs, einv, eoutv, kev, rows, semi, semg):
        cid = lax.axis_index("c")
        sid = lax.axis_index("s")
        wid = cid * 16 + sid
        base0 = wid * ew
        r0 = sid * rows_per_tile

        zb = rows[0]
        def zrow(i, _):
            zb[i, pl.ds(0, 16)] = jnp.zeros((16,), jnp.float32)
            zb[i, pl.ds(16, 16)] = jnp.zeros((16,), jnp.float32)
            return _
        lax.fori_loop(0, zc, zrow, None)

        def zcopy(i, _):
            pltpu.sync_copy(zb.at[pl.ds(0, zc)], aggs.at[pl.ds(r0 + i * zc, zc)])
            return _
        lax.fori_loop(0, 14, zcopy, None)
        plsc.subcore_barrier()

        def fire_idx(i, p):
            base = base0 + i * ca
            rb = base // CHUNK
            pltpu.async_copy(ein2.at[pl.ds(rb, 2)], einv[p], semi[p])
            pltpu.async_copy(eout2.at[pl.ds(rb, 2)], eoutv[p], semi[p])
            pltpu.async_copy(ke.at[pl.ds(base, ca)], kev[p], semi[p])

        def wait_idx(p):
            pltpu.make_async_copy(ein2.at[pl.ds(0, 2)], einv[p], semi[p]).wait()
            pltpu.make_async_copy(ein2.at[pl.ds(0, 2)], eoutv[p], semi[p]).wait()
            pltpu.make_async_copy(ke.at[pl.ds(0, ca)], kev[p], semi[p]).wait()

        def fire_gather(p):
            for j in range(2):
                pltpu.async_copy(qp.at[einv[p].at[j]],
                                 rows[p].at[pl.ds(j * CHUNK, CHUNK)], semg[p])

        def wait_gather(p):
            for j in range(2):
                pltpu.make_async_copy(qp.at[einv[p].at[j]],
                                      rows[p].at[pl.ds(j * CHUNK, CHUNK)],
                                      semg[p]).wait()

        fire_idx(0, 0)
        fire_idx(1, 1)
        wait_idx(0)
        fire_gather(0)

        def step(i, par):
            @pl.when(i + 1 < nchunks)
            def _():
                wait_idx(1 - par)
                fire_gather(1 - par)
            wait_gather(par)
            rw = rows[par]
            kw = kev[par]
            for gidx in range(ca // 16):
                kv = kw[pl.ds(gidx * 16, 16)]
                for j in range(16):
                    e = gidx * 16 + j
                    s = _take16(kv, j)
                    rw[e, pl.ds(0, 16)] = rw[e, pl.ds(0, 16)] * s
                    rw[e, pl.ds(16, 16)] = rw[e, pl.ds(16, 16)] * s
            for j in range(2):
                pltpu.sync_copy(rw.at[pl.ds(j * CHUNK, CHUNK)],
                                aggs.at[eoutv[par].at[j]], add=True)

            @pl.when(i + 2 < nchunks)
            def _():
                fire_idx(i + 2, par)

        def body2(i2, _):
            step(i2 * 2, 0)
            step(i2 * 2 + 1, 1)
            return _

        lax.fori_loop(0, nchunks // 2, body2, None)
        plsc.subcore_barrier()

        def ocopy(i, _):
            pltpu.sync_copy(aggs.at[pl.ds(r0 + i * zc, zc)],
                            agg_out.at[cid, pl.ds(r0 + i * zc, zc)])
            return _
        lax.fori_loop(0, 14, ocopy, None)

    return agg_kernel


def kernel(feat, img, dist_diff, msg_node, alpha, beta, gamma, w1, w2):
    B, K, H, W = feat.shape
    N = B * H * W
    E = msg_node.shape[0]
    MAX_IT = 10

    featp = jnp.pad(feat.reshape(K, N).T, ((0, 0), (0, D - K)))
    imgc = img.reshape(3, N)
    ein = msg_node[:, 0]
    eout = msg_node[:, 1]
    d0 = dist_diff[:, 0]
    d1 = dist_diff[:, 1]
    params = jnp.concatenate([alpha.ravel(), beta.ravel(), gamma.ravel(),
                              w1.ravel(), w2.ravel(),
                              jnp.ones((7,), jnp.float32)])

    RB = 3584
    grid = (N // RB,)

    q0, unary = pl.pallas_call(
        functools.partial(_init_body, K),
        grid=grid,
        in_specs=[pl.BlockSpec((RB, D), lambda i: (i, 0))],
        out_specs=[pl.BlockSpec((RB, D), lambda i: (i, 0))] * 2,
        out_shape=[jax.ShapeDtypeStruct((N, D), jnp.float32)] * 2,
    )(featp)

    ke = _edge_coef_kernel(N, E)(imgc[0], imgc[1], imgc[2],
                                 ein, eout, d0, d1, params)

    ein2 = ein.reshape(E // CHUNK, CHUNK)
    eout2 = eout.reshape(E // CHUNK, CHUNK)

    def update(log_out, agg2, un):
        return pl.pallas_call(
            functools.partial(_update_body, K, log_out),
            grid=grid,
            in_specs=[pl.BlockSpec((2, RB, D), lambda i: (0, i, 0)),
                      pl.BlockSpec((RB, D), lambda i: (i, 0))],
            out_specs=pl.BlockSpec((RB, D), lambda i: (i, 0)),
            out_shape=jax.ShapeDtypeStruct((N, D), jnp.float32),
        )(agg2, un)

    agg2, _qf, _xb, _fl = _fused_kernel(N, E, MAX_IT)(
        q0, ein2, eout2, ke, unary)
    q = update(True, agg2, unary)

    logq = q[:, :K].reshape(B, H, W, K)
    return jnp.transpose(logq, (0, 3, 1, 2))


# speedup vs baseline: 22.6323x; 1.0002x over previous
import functools

import jax
import jax.numpy as jnp
from jax import lax
from jax.experimental import pallas as pl
from jax.experimental.pallas import tpu as pltpu
from jax.experimental.pallas import tpu_sc as plsc

D = 32
DI = 16
CHUNK = 128
NW = 32
BIG = 1.0e30


def _take16(v, j):
    idx = jnp.full((16, 1), j, dtype=jnp.int32)
    dnums = lax.GatherDimensionNumbers(
        offset_dims=(), collapsed_slice_dims=(0,), start_index_map=(0,))
    return lax.gather(v, idx, dnums, (1,),
                      mode=lax.GatherScatterMode.PROMISE_IN_BOUNDS)


def _rot16(v, sh):
    idx = ((lax.iota(jnp.int32, 16) + sh) % 16).reshape(16, 1)
    dnums = lax.GatherDimensionNumbers(
        offset_dims=(), collapsed_slice_dims=(0,), start_index_map=(0,))
    return lax.gather(v, idx, dnums, (1,),
                      mode=lax.GatherScatterMode.PROMISE_IN_BOUNDS)


def _vsum16(v):
    for sh in (8, 4, 2, 1):
        v = v + _rot16(v, sh)
    return v


def _lane0(v):
    return lax.squeeze(lax.slice(v, (0,), (1,)), (0,))


def _init_body(k, featp_ref, q0_ref, un_ref):
    x = featp_ref[...]
    lane = lax.broadcasted_iota(jnp.int32, x.shape, 1)
    xm = jnp.where(lane < k, x, -BIG)
    m = jnp.max(xm, axis=1, keepdims=True)
    e = jnp.exp(xm - m)
    s = jnp.sum(e, axis=1, keepdims=True)
    q0_ref[...] = e / s
    un_ref[...] = jnp.where(lane < k, m + jnp.log(s) - xm, BIG)


def _wvec(k, shape):
    lane = lax.broadcasted_iota(jnp.int32, shape, 1)
    return jnp.where(lane == k - 1, 10.0,
                     jnp.where(lane < k, 1.0, 0.0)).astype(jnp.float32)


def _update_body(k, log_out, agg_ref, un_ref, out_ref):
    agg = agg_ref[0] + agg_ref[1]
    w = _wvec(k, agg.shape)
    s = jnp.sum(agg * w, axis=1, keepdims=True)
    logit = w * w * agg - w * s - un_ref[...]
    m = jnp.max(logit, axis=1, keepdims=True)
    e = jnp.exp(logit - m)
    z = jnp.sum(e, axis=1, keepdims=True)
    if log_out:
        out_ref[...] = logit - m - jnp.log(z)
    else:
        out_ref[...] = e / z


def _edge_coef_kernel(n, e_total):
    ew = e_total // NW
    nchunks = ew // CHUNK
    mesh = plsc.VectorSubcoreMesh(core_axis_name="c", subcore_axis_name="s", num_cores=2, num_subcores=16)

    @functools.partial(
        pl.kernel,
        out_type=jax.ShapeDtypeStruct((e_total,), jnp.float32),
        mesh=mesh,
        compiler_params=pltpu.CompilerParams(use_tc_tiling_on_sc=False),
        scratch_types=[
            [pltpu.VMEM((CHUNK,), jnp.int32)] * 2,
            [pltpu.VMEM((CHUNK,), jnp.int32)] * 2,
            [pltpu.VMEM((CHUNK,), jnp.float32)] * 2,
            [pltpu.VMEM((CHUNK,), jnp.float32)] * 2,
            [[pltpu.VMEM((CHUNK,), jnp.float32)] * 3] * 2,
            [[pltpu.VMEM((CHUNK,), jnp.float32)] * 3] * 2,
            pltpu.VMEM((CHUNK,), jnp.float32),
            pltpu.VMEM((16,), jnp.float32),
            [pltpu.SemaphoreType.DMA] * 2,
            [pltpu.SemaphoreType.DMA] * 2,
        ],
    )
    def coef_kernel(ch0, ch1, ch2, ein, eout, d0, d1, params, ke_out,
                    einv, eoutv, d0v, d1v, rin, rout, kev, prmv, semi, semg):
        wid = lax.axis_index("c") * 16 + lax.axis_index("s")
        base0 = wid * ew
        pltpu.sync_copy(params, prmv)
        praw = prmv[...]
        pos = lax.iota(jnp.int32, 16)
        cv = jnp.where(pos < 7, 1.0 / (2.0 * praw * praw), jnp.abs(praw))
        a0 = _take16(cv, 0)
        a1 = _take16(cv, 1)
        b = [_take16(cv, 2), _take16(cv, 3), _take16(cv, 4)]
        g0 = _take16(cv, 5)
        g1 = _take16(cv, 6)
        aw1 = _take16(cv, 7)
        aw2 = _take16(cv, 8)
        chans = (ch0, ch1, ch2)

        def fire_idx(i, p):
            base = base0 + i * CHUNK
            pltpu.async_copy(ein.at[pl.ds(base, CHUNK)], einv[p], semi[p])
            pltpu.async_copy(eout.at[pl.ds(base, CHUNK)], eoutv[p], semi[p])
            pltpu.async_copy(d0.at[pl.ds(base, CHUNK)], d0v[p], semi[p])
            pltpu.async_copy(d1.at[pl.ds(base, CHUNK)], d1v[p], semi[p])

        def wait_idx(p):
            for dst in (einv[p], eoutv[p], d0v[p], d1v[p]):
                pltpu.make_async_copy(d0.at[pl.ds(0, CHUNK)], dst, semi[p]).wait()

        def fire_gather(p):
            for ch in range(3):
                pltpu.async_copy(chans[ch].at[einv[p]], rin[p][ch], semg[p])
                pltpu.async_copy(chans[ch].at[eoutv[p]], rout[p][ch], semg[p])

        def wait_gather(p):
            for ch in range(3):
                pltpu.make_async_copy(ch0.at[einv[p]], rin[p][ch], semg[p]).wait()
                pltpu.make_async_copy(ch0.at[einv[p]], rout[p][ch], semg[p]).wait()

        fire_idx(0, 0)
        fire_idx(1, 1)
        wait_idx(0)
        fire_gather(0)

        def step(i, par):
            @pl.when(i + 1 < nchunks)
            def _():
                wait_idx(1 - par)
                fire_gather(1 - par)
            wait_gather(par)
            for gidx in range(CHUNK // 16):
                sl = pl.ds(gidx * 16, 16)
                acc = jnp.zeros((16,), jnp.float32)
                for ch in range(3):
                    dd = rin[par][ch][sl] - rout[par][ch][sl]
                    acc = acc + dd * dd * b[ch]
                dv0 = d0v[par][sl]
                dv1 = d1v[par][sl]
                k1 = aw1 * jnp.exp(-(dv0 * a0 + dv1 * a1) - acc)
                k2 = aw2 * jnp.exp(-(dv0 * g0 + dv1 * g1))
                kev[sl] = k1 + k2
            base = base0 + i * CHUNK
            pltpu.sync_copy(kev, ke_out.at[pl.ds(base, CHUNK)])

            @pl.when(i + 2 < nchunks)
            def _():
                fire_idx(i + 2, par)

        def body2(i2, _):
            step(i2 * 2, 0)
            step(i2 * 2 + 1, 1)
            return _

        lax.fori_loop(0, nchunks // 2, body2, None)

    return coef_kernel


def _fused_kernel(n, e_total, max_it):
    ew = e_total // NW
    ca = 2 * CHUNK
    nchunks = ew // ca
    rpt = n // 16
    half = n // 2
    rhalf = n // 32
    zc = rpt // 14
    sm = 112
    nsm = rhalf // sm
    mesh = plsc.VectorSubcoreMesh(core_axis_name="c", subcore_axis_name="s", num_cores=2, num_subcores=16)

    @functools.partial(
        pl.kernel,
        out_type=[jax.ShapeDtypeStruct((2, n, D), jnp.float32),
                  jax.ShapeDtypeStruct((n, D), jnp.float32),
                  jax.ShapeDtypeStruct((2, half, D), jnp.float32),
                  jax.ShapeDtypeStruct((2, 20, 16), jnp.float32)],
        mesh=mesh,
        compiler_params=pltpu.CompilerParams(use_tc_tiling_on_sc=False),
        scratch_types=[
            pltpu.VMEM_SHARED((n, D), jnp.float32),
            [pltpu.VMEM((2, CHUNK), jnp.int32)] * 2,
            [pltpu.VMEM((2, CHUNK), jnp.int32)] * 2,
            [pltpu.VMEM((ca,), jnp.float32)] * 2,
            [pltpu.VMEM((ca, D), jnp.float32)] * 2,
            pltpu.VMEM((16,), jnp.float32),
            pltpu.VMEM((16,), jnp.float32),
            pltpu.VMEM((20, 16), jnp.float32),
            [pltpu.SemaphoreType.DMA] * 2,
            [pltpu.SemaphoreType.DMA] * 2,
        ],
    )
    def fused_kernel(q0, ein2, eout2, ke, unary,
                     agg_out, qcur, xbuf, flags,
                     aggs, einv, eoutv, kev, rows, fones, fbuf, zf, semi, semg):
        cid = lax.axis_index("c")
        sid = lax.axis_index("s")
        wid = cid * 16 + sid
        base0 = wid * ew
        r0 = sid * rpt
        oth = 1 - cid

        fones[...] = jnp.ones((16,), jnp.float32)

        @pl.when(sid == 0)
        def _():
            def zfrow(i, _):
                zf[i, pl.ds(0, 16)] = jnp.zeros((16,), jnp.float32)
                return _
            lax.fori_loop(0, 20, zfrow, None)
            pltpu.sync_copy(zf, flags.at[cid])

        def wflag(ev):
            @pl.when(sid == 0)
            def _():
                pltpu.sync_copy(fones, flags.at[cid, ev])

        def poll(ev):
            fbuf[...] = jnp.zeros((16,), jnp.float32)

            def pbody(k, done):
                @pl.when(done < 0.5)
                def _():
                    pltpu.sync_copy(flags.at[oth, ev], fbuf)
                return jnp.maximum(done, _lane0(fbuf[...]))

            d = lax.fori_loop(0, 256, pbody, jnp.float32(0))

            @pl.when(d < 0.5)
            def _():
                lax.fori_loop(0, 8192, pbody, jnp.float32(0))

        def fill_zeros():
            def zrow(i, _):
                rows[0][i, pl.ds(0, 16)] = jnp.zeros((16,), jnp.float32)
                rows[0][i, pl.ds(16, 16)] = jnp.zeros((16,), jnp.float32)
                return _
            lax.fori_loop(0, zc, zrow, None)

        def zero_aggs():
            def zcopy(i, _):
                pltpu.sync_copy(rows[0].at[pl.ds(0, zc)],
                                aggs.at[pl.ds(r0 + i * zc, zc)])
                return _
            lax.fori_loop(0, 14, zcopy, None)

        fill_zeros()
        zero_aggs()

        def icopy(i, _):
            pltpu.sync_copy(q0.at[pl.ds(r0 + i * zc, zc)], rows[1].at[pl.ds(0, zc)])
            pltpu.sync_copy(rows[1].at[pl.ds(0, zc)], qcur.at[pl.ds(r0 + i * zc, zc)])
            return _
        lax.fori_loop(0, 14, icopy, None)
        plsc.subcore_barrier()
        wflag(0)
        poll(0)

        def fire_idx(i, p):
            base = base0 + i * ca
            rb = base // CHUNK
            pltpu.async_copy(ein2.at[pl.ds(rb, 2)], einv[p], semi[p])
            pltpu.async_copy(eout2.at[pl.ds(rb, 2)], eoutv[p], semi[p])
            pltpu.async_copy(ke.at[pl.ds(base, ca)], kev[p], semi[p])

        def wait_idx(p):
            pltpu.make_async_copy(ein2.at[pl.ds(0, 2)], einv[p], semi[p]).wait()
            pltpu.make_async_copy(ein2.at[pl.ds(0, 2)], eoutv[p], semi[p]).wait()
            pltpu.make_async_copy(ke.at[pl.ds(0, ca)], kev[p], semi[p]).wait()

        def fire_gather(p):
            for j in range(2):
                pltpu.async_copy(qcur.at[einv[p].at[j]],
                                 rows[p].at[pl.ds(j * CHUNK, CHUNK)], semg[p])

        def wait_gather(p):
            for j in range(2):
                pltpu.make_async_copy(qcur.at[einv[p].at[j]],
                                      rows[p].at[pl.ds(j * CHUNK, CHUNK)],
                                      semg[p]).wait()

        def edge_phase():
            fire_idx(0, 0)
            fire_idx(1, 1)
            wait_idx(0)
            fire_gather(0)

            def step(i, par):
                @pl.when(i + 1 < nchunks)
                def _():
                    wait_idx(1 - par)
                    fire_gather(1 - par)
                wait_gather(par)
                rw = rows[par]
                kw = kev[par]

                def gloop(g, _):
                    kv = kw[pl.ds(g * 16, 16)]
                    for j in range(16):
                        e = g * 16 + j
                        s = _take16(kv, j)
                        rw[e, pl.ds(0, 16)] = rw[e, pl.ds(0, 16)] * s
                        rw[e, pl.ds(16, 16)] = rw[e, pl.ds(16, 16)] * s
                    return _
                lax.fori_loop(0, ca // 16, gloop, None)
                for j in range(2):
                    pltpu.sync_copy(rw.at[pl.ds(j * CHUNK, CHUNK)],
                                    aggs.at[eoutv[par].at[j]], add=True)

                @pl.when(i + 2 < nchunks)
                def _():
                    fire_idx(i + 2, par)

            def body2(i2, _):
                step(i2 * 2, 0)
                step(i2 * 2 + 1, 1)
                return _
            lax.fori_loop(0, nchunks // 2, body2, None)

        w1v = jnp.where(lax.iota(jnp.int32, 16) == 4, 10.0,
                        jnp.where(lax.iota(jnp.int32, 16) < 5, 1.0, 0.0)
                        ).astype(jnp.float32)

        def sm_phase():
            def smchunk(i, _):
                rb = cid * half + sid * rhalf + i * sm
                rel = sid * rhalf + i * sm
                pltpu.sync_copy(aggs.at[pl.ds(rb, sm)],
                                rows[0].at[pl.ds(0, sm)])
                pltpu.sync_copy(xbuf.at[oth, pl.ds(rel, sm)],
                                rows[0].at[pl.ds(sm, sm)])
                pltpu.sync_copy(unary.at[pl.ds(rb, sm)],
                                rows[1].at[pl.ds(0, sm)])

                def node(r, _):
                    rx = sm + r
                    a0 = rows[0][r, pl.ds(0, 16)] + rows[0][rx, pl.ds(0, 16)]
                    a1 = rows[0][r, pl.ds(16, 16)] + rows[0][rx, pl.ds(16, 16)]
                    u0 = rows[1][r, pl.ds(0, 16)]
                    u1 = rows[1][r, pl.ds(16, 16)]
                    s = _vsum16(a0 + a1 * w1v)
                    l0 = a0 - s - u0
                    l1 = w1v * w1v * a1 - w1v * s - u1
                    e0 = jnp.exp(l0)
                    e1 = jnp.exp(l1)
                    rz = 1.0 / _vsum16(e0 + e1)
                    rows[1][rx, pl.ds(0, 16)] = e0 * rz
                    rows[1][rx, pl.ds(16, 16)] = e1 * rz
                    return _
                lax.fori_loop(0, sm, node, None)
                pltpu.sync_copy(rows[1].at[pl.ds(sm, sm)],
                                qcur.at[pl.ds(rb, sm)])
                return _
            lax.fori_loop(0, nsm, smchunk, None)

        for t in range(max_it):
            if t > 0:
                poll(2 * t)
            edge_phase()
            plsc.subcore_barrier()

            if t < max_it - 1:
                in_oth = (r0 >= oth * half) & (r0 < (oth + 1) * half)

                @pl.when(in_oth)
                def _():
                    def xcopy(i, _):
                        pltpu.sync_copy(
                            aggs.at[pl.ds(r0 + i * zc, zc)],
                            xbuf.at[cid, pl.ds(r0 - oth * half + i * zc, zc)])
                        return _
                    lax.fori_loop(0, 14, xcopy, None)
                plsc.subcore_barrier()
                wflag(1 + 2 * t)
                poll(1 + 2 * t)
                sm_phase()
                plsc.subcore_barrier()
                wflag(2 + 2 * t)
                fill_zeros()
                zero_aggs()
                plsc.subcore_barrier()
            else:
                def ocopy(i, _):
                    pltpu.sync_copy(aggs.at[pl.ds(r0 + i * zc, zc)],
                                    agg_out.at[cid, pl.ds(r0 + i * zc, zc)])
                    return _
                lax.fori_loop(0, 14, ocopy, None)

    return fused_kernel


def kernel(feat, img, dist_diff, msg_node, alpha, beta, gamma, w1, w2):
    B, K, H, W = feat.shape
    N = B * H * W
    E = msg_node.shape[0]
    MAX_IT = 10

    featp = jnp.pad(feat.reshape(K, N).T, ((0, 0), (0, D - K)))
    imgc = img.reshape(3, N)
    ein = msg_node[:, 0]
    eout = msg_node[:, 1]
    d0 = dist_diff[:, 0]
    d1 = dist_diff[:, 1]
    params = jnp.concatenate([alpha.ravel(), beta.ravel(), gamma.ravel(),
                              w1.ravel(), w2.ravel(),
                              jnp.ones((7,), jnp.float32)])

    RB = 3584
    grid = (N // RB,)

    q0, unary = pl.pallas_call(
        functools.partial(_init_body, K),
        grid=grid,
        in_specs=[pl.BlockSpec((RB, D), lambda i: (i, 0))],
        out_specs=[pl.BlockSpec((RB, D), lambda i: (i, 0))] * 2,
        out_shape=[jax.ShapeDtypeStruct((N, D), jnp.float32)] * 2,
    )(featp)

    ke = _edge_coef_kernel(N, E)(imgc[0], imgc[1], imgc[2],
                                 ein, eout, d0, d1, params)

    ein2 = ein.reshape(E // CHUNK, CHUNK)
    eout2 = eout.reshape(E // CHUNK, CHUNK)

    def update(log_out, agg2, un):
        return pl.pallas_call(
            functools.partial(_update_body, K, log_out),
            grid=grid,
            in_specs=[pl.BlockSpec((2, RB, D), lambda i: (0, i, 0)),
                      pl.BlockSpec((RB, D), lambda i: (i, 0))],
            out_specs=pl.BlockSpec((RB, D), lambda i: (i, 0)),
            out_shape=jax.ShapeDtypeStruct((N, D), jnp.float32),
        )(agg2, un)

    agg2, _qf, _xb, _fl = _fused_kernel(N, E, MAX_IT)(
        q0, ein2, eout2, ke, unary)
    q = update(True, agg2, unary)

    logq = q[:, :K].reshape(B, H, W, K)
    return jnp.transpose(logq, (0, 3, 1, 2))
